# baseline (speedup 1.0000x reference)
import jax
import jax.numpy as jnp
from jax import lax
from jax.experimental import pallas as pl
from jax.experimental.pallas import tpu as pltpu
from jax.experimental.pallas import tpu_sc as plsc

BRANCH = 10
DEPTH = 5
N = 111111
BATCH = 256
_OFF = [(BRANCH**i - 1) // (BRANCH - 1) for i in range(DEPTH + 2)]

_NC, _NS = 2, 16
_NW = _NC * _NS
_ROWS_PER_W = BATCH // _NW


def _level_reduce(buf, parent_base, child_base, n_parents):
    lane = lax.iota(jnp.int32, 16)
    n_full = n_parents // 16
    rem = n_parents % 16

    def step(i, masked):
        p_rel = i * 16 + lane
        m = buf[pl.ds(parent_base + i * 16, 16)]
        cbase = child_base + p_rel * BRANCH
        for j in range(BRANCH):
            m = jnp.maximum(m, plsc.load_gather(buf, [cbase + j]))
        if masked:
            plsc.store_scatter(buf, [parent_base + p_rel], m,
                               mask=p_rel < n_parents)
        else:
            buf[pl.ds(parent_base + i * 16, 16)] = m

    if n_full > 0:
        def body(i, carry):
            step(i, False)
            return carry
        lax.fori_loop(0, n_full, body, 0)
    if rem:
        step(n_full, True)


def _sc_body(x_hbm, out_hbm, buf):
    wid = lax.axis_index("s") * _NC + lax.axis_index("c")

    def row_body(r, carry):
        row = wid * _ROWS_PER_W + r
        pltpu.sync_copy(x_hbm.at[row], buf)
        for lvl in reversed(range(DEPTH)):
            _level_reduce(buf, _OFF[lvl], _OFF[lvl + 1], BRANCH**lvl)
        pltpu.sync_copy(buf, out_hbm.at[row])
        return carry

    lax.fori_loop(0, _ROWS_PER_W, row_body, 0)


def kernel(x, level_parents, level_children):
    del level_parents, level_children
    fn = pl.kernel(
        _sc_body,
        out_type=jax.ShapeDtypeStruct((BATCH, N), jnp.float32),
        mesh=plsc.VectorSubcoreMesh(core_axis_name="c", subcore_axis_name="s"),
        scratch_types=[pltpu.VMEM((N,), jnp.float32)],
        compiler_params=pltpu.CompilerParams(use_tc_tiling_on_sc=False,
                                             needs_layout_passes=False),
    )
    return fn(x)


# speedup vs baseline: 14.3751x; 14.3751x over previous
import jax
import jax.numpy as jnp
from jax import lax
from jax.experimental import pallas as pl
from jax.experimental.pallas import tpu as pltpu
from jax.experimental.pallas import tpu_sc as plsc

N = 111111
B = 256
H = 128

_CP = pltpu.CompilerParams(needs_layout_passes=False)
_MESH = dict(core_axis_name="c", subcore_axis_name="s")


def _fold_group(dst, di, own, oi, ch, cb, nch=10):
    for v in range(8):
        sl = pl.ds(v * 16, 16)
        m = own[oi, sl]
        for k in range(nch):
            m = jnp.maximum(m, ch[cb + k, sl])
        dst[di, sl] = m


def _k1_body(xt, y, buf_a, buf_b, big, obuf, rbuf, sem_a, sem_b, sem_ao, sem_bo):
    c = lax.axis_index("c")
    s = lax.axis_index("s")
    w = c * 16 + s
    r0 = 1112 + 312 * w
    rdbase = 10 * r0

    bufs = (buf_a, buf_b)
    isems = (sem_a, sem_b)
    osems = (sem_ao, sem_bo)

    for h in range(2):
        col = pl.ds(h * H, H)

        pltpu.sync_copy(xt.at[pl.ds(r0, 312), col], big.at[pl.ds(0, 312)])

        def issue_in(i, k):
            pltpu.async_copy(xt.at[pl.ds(rdbase + 120 * k, 128), col],
                             bufs[i], isems[i])

        def wait_in(i):
            pltpu.make_async_copy(xt.at[pl.ds(0, 128), col],
                                  bufs[i], isems[i]).wait()

        def issue_out(i, k):
            pltpu.async_copy(bufs[i].at[pl.ds(8, 120)],
                             y.at[pl.ds(rdbase + 8 + 120 * k, 120), col],
                             osems[i])

        def wait_out(i):
            pltpu.make_async_copy(bufs[i].at[pl.ds(8, 120)],
                                  y.at[pl.ds(11128, 120), col],
                                  osems[i]).wait()

        def compute(i, k):
            def gbody(g, carry):
                _fold_group(big, 12 * k + g, big, 12 * k + g, bufs[i],
                            1 + 10 * g)
                return carry
            lax.fori_loop(0, 12, gbody, 0)

        def chunk(i, k):
            wait_in(i)
            issue_out(i, k)
            compute(i, k)
            wait_out(i)

        issue_in(0, 0)
        issue_in(1, 1)

        def loop_t(t, carry):
            chunk(0, 2 * t)
            issue_in(0, 2 * t + 2)
            chunk(1, 2 * t + 1)
            issue_in(1, 2 * t + 3)
            return carry

        lax.fori_loop(0, 12, loop_t, 0)
        chunk(0, 24)
        chunk(1, 25)

        pltpu.sync_copy(big.at[pl.ds(0, 312)], y.at[pl.ds(r0, 312), col])

    plsc.subcore_barrier()

    qn = 496 + 8 * c
    q0 = 112 + 496 * c
    qa = q0 + jnp.minimum(32 * s, qn - 32)

    for h in range(2):
        col = pl.ds(h * H, H)
        pltpu.sync_copy(y.at[pl.ds(10 * qa, 328), col], big.at[pl.ds(0, 328)])
        pltpu.sync_copy(xt.at[pl.ds(qa, 32), col], obuf)

        def l3body(g, carry):
            _fold_group(rbuf, g, obuf, g, big, 1 + 10 * g)
            return carry
        lax.fori_loop(0, 32, l3body, 0)

        @pl.when(jnp.logical_and(c == 1, s == 0))
        def _special():
            pltpu.sync_copy(xt.at[pl.ds(6080, 48), col], buf_b.at[pl.ds(0, 48)])
            for t in range(3):
                a_t = (60808, 60904, 61008)[t]
                d_t = (60811 + 100 * t) - a_t
                sz = (104, 112, 104)[t]
                pltpu.sync_copy(xt.at[pl.ds(a_t, sz), col],
                                buf_a.at[pl.ds(0, sz)])
                for v in range(8):
                    sl = pl.ds(v * 16, 16)
                    rbuf[t, sl] = obuf[t, sl]

                def pbody(p, carry):
                    for v in range(8):
                        sl = pl.ds(v * 16, 16)
                        pv = buf_b[1 + 10 * t + p, sl]
                        for k in range(10):
                            pv = jnp.maximum(pv, buf_a[d_t + 10 * p + k, sl])
                        rbuf[t, sl] = jnp.maximum(rbuf[t, sl], pv)
                    return carry
                lax.fori_loop(0, 10, pbody, 0)

        pltpu.sync_copy(rbuf, y.at[pl.ds(qa, 32), col])


def _k2_body(xt, y, cbuf, xbuf, obuf, sbuf_s, sbuf_l, sbuf_t, fixb):
    c = lax.axis_index("c")
    s = lax.axis_index("s")
    w = c * 16 + s

    @pl.when(w == 0)
    def _w0():
        for h in range(2):
            col = pl.ds(h * H, H)

            pltpu.sync_copy(xt.at[pl.ds(11096, 16), col], sbuf_s)
            pltpu.sync_copy(xt.at[pl.ds(110960, 151), col],
                            sbuf_l.at[pl.ds(0, 151)])

            def pt(p, carry):
                _fold_group(sbuf_s, p, sbuf_s, p, sbuf_l, 1 + 10 * p)
                return carry
            lax.fori_loop(0, 15, pt, 0)
            pltpu.sync_copy(sbuf_s, y.at[pl.ds(11096, 16), col])

            pltpu.sync_copy(xt.at[pl.ds(11104, 24), col], sbuf_t)
            pltpu.sync_copy(sbuf_t.at[pl.ds(8, 16)],
                            y.at[pl.ds(11112, 16), col])
            pltpu.sync_copy(sbuf_l.at[pl.ds(8, 136)],
                            y.at[pl.ds(110968, 136), col])
            pltpu.sync_copy(sbuf_l.at[pl.ds(144, 7)],
                            y.at[pl.ds(111104, 7), col])

            pltpu.sync_copy(y.at[pl.ds(11088, 8), col], fixb.at[pl.ds(8, 8)])
            pltpu.sync_copy(y.at[pl.ds(1104, 8), col], fixb.at[pl.ds(0, 8)])
            pltpu.sync_copy(xt.at[pl.ds(1104, 8), col], fixb.at[pl.ds(16, 8)])
            for v in range(8):
                sl = pl.ds(v * 16, 16)
                m = fixb[16 + 5, sl]
                for k in range(5):
                    m = jnp.maximum(m, fixb[8 + 3 + k, sl])
                for k in range(5):
                    m = jnp.maximum(m, sbuf_s[k, sl])
                fixb[5, sl] = m
                m = fixb[16 + 6, sl]
                for k in range(10):
                    m = jnp.maximum(m, sbuf_s[5 + k, sl])
                fixb[6, sl] = m
                m = jnp.maximum(fixb[16 + 7, sl], sbuf_s[15, sl])
                for k in range(9):
                    m = jnp.maximum(m, sbuf_t[8 + k, sl])
                fixb[7, sl] = m
            pltpu.sync_copy(fixb.at[pl.ds(0, 8)], y.at[pl.ds(1104, 8), col])

            pltpu.sync_copy(xt.at[pl.ds(0, 112), col], xbuf)

            pltpu.sync_copy(y.at[pl.ds(608, 520), col], cbuf)

            def l2b(g, carry):
                _fold_group(obuf, 61 + g, xbuf, 61 + g, cbuf, 3 + 10 * g)
                return carry
            lax.fori_loop(0, 50, l2b, 0)
            _fold_group(obuf, 111, xbuf, 111, cbuf, 503)

            pltpu.sync_copy(y.at[pl.ds(104, 512), col], cbuf.at[pl.ds(0, 512)])

            def l2a(g, carry):
                _fold_group(obuf, 11 + g, xbuf, 11 + g, cbuf, 7 + 10 * g)
                return carry
            lax.fori_loop(1, 50, l2a, 0)
            for v in range(8):
                sl = pl.ds(v * 16, 16)
                m = jnp.maximum(xbuf[11, sl], obuf[111, sl])
                for k in range(9):
                    m = jnp.maximum(m, cbuf[8 + k, sl])
                obuf[11, sl] = m

            def l1(r, carry):
                _fold_group(obuf, r, xbuf, r, obuf, 10 * r + 1)
                return carry
            lax.fori_loop(1, 11, l1, 0)

            _fold_group(obuf, 0, xbuf, 0, obuf, 1)

            pltpu.sync_copy(obuf, y.at[pl.ds(0, 112), col])


def kernel(x, level_parents, level_children):
    del level_parents, level_children
    xt = x.T

    k1 = pl.kernel(
        _k1_body, out_type=(),
        mesh=plsc.VectorSubcoreMesh(**_MESH),
        scratch_types=[
            pltpu.VMEM((128, H), jnp.float32),
            pltpu.VMEM((128, H), jnp.float32),
            pltpu.VMEM((328, H), jnp.float32),
            pltpu.VMEM((32, H), jnp.float32),
            pltpu.VMEM((32, H), jnp.float32),
            pltpu.SemaphoreType.DMA,
            pltpu.SemaphoreType.DMA,
            pltpu.SemaphoreType.DMA,
            pltpu.SemaphoreType.DMA,
        ],
        compiler_params=_CP,
    )
    k2 = pl.kernel(
        _k2_body, out_type=(),
        mesh=plsc.VectorSubcoreMesh(**_MESH),
        scratch_types=[
            pltpu.VMEM((520, H), jnp.float32),
            pltpu.VMEM((112, H), jnp.float32),
            pltpu.VMEM((112, H), jnp.float32),
            pltpu.VMEM((16, H), jnp.float32),
            pltpu.VMEM((152, H), jnp.float32),
            pltpu.VMEM((24, H), jnp.float32),
            pltpu.VMEM((24, H), jnp.float32),
        ],
        compiler_params=_CP,
    )

    y_ref = jax.new_ref(lax.empty((N, B), jnp.float32))
    k1(xt, y_ref)
    k2(xt, y_ref)
    return y_ref[...].T


# speedup vs baseline: 16.0027x; 1.1132x over previous
import jax
import jax.numpy as jnp
from jax import lax
from jax.experimental import pallas as pl
from jax.experimental.pallas import tpu as pltpu
from jax.experimental.pallas import tpu_sc as plsc

N = 111111
B = 256
H = 128

_CP = pltpu.CompilerParams(needs_layout_passes=False)
_MESH = dict(core_axis_name="c", subcore_axis_name="s")


def _fold_group(dst, di, own, oi, ch, cb, nch=10):
    for v in range(8):
        sl = pl.ds(v * 16, 16)
        m = own[oi, sl]
        for k in range(nch):
            m = jnp.maximum(m, ch[cb + k, sl])
        dst[di, sl] = m


def _k1_body(xt, y, buf_a, buf_b, buf_c, big, obuf, rbuf,
             sem_a, sem_b, sem_c, sem_ao, sem_bo, sem_co):
    c = lax.axis_index("c")
    s = lax.axis_index("s")
    w = c * 16 + s
    r0 = 1112 + 312 * w
    rdbase = 10 * r0

    bufs = (buf_a, buf_b, buf_c)
    isems = (sem_a, sem_b, sem_c)
    osems = (sem_ao, sem_bo, sem_co)

    for h in range(2):
        col = pl.ds(h * H, H)

        pltpu.sync_copy(xt.at[pl.ds(r0, 312), col], big.at[pl.ds(0, 312)])

        def issue_in(i, k):
            pltpu.async_copy(xt.at[pl.ds(rdbase + 120 * k, 128), col],
                             bufs[i], isems[i])

        def wait_in(i):
            pltpu.make_async_copy(xt.at[pl.ds(0, 128), col],
                                  bufs[i], isems[i]).wait()

        def issue_out(i, k):
            pltpu.async_copy(bufs[i].at[pl.ds(8, 120)],
                             y.at[pl.ds(rdbase + 8 + 120 * k, 120), col],
                             osems[i])

        def wait_out(i):
            pltpu.make_async_copy(bufs[i].at[pl.ds(8, 120)],
                                  y.at[pl.ds(11128, 120), col],
                                  osems[i]).wait()

        def compute(i, k):
            def gbody(g, carry):
                _fold_group(big, 12 * k + g, big, 12 * k + g, bufs[i],
                            1 + 10 * g)
                return carry
            lax.fori_loop(0, 12, gbody, 0)

        def chunk(i, k, wait_prev, issue_next):
            wait_in(i)
            issue_out(i, k)
            compute(i, k)
            if wait_prev:
                wait_out((i + 2) % 3)
            if issue_next:
                issue_in((i + 2) % 3, k + 2)

        issue_in(0, 0)
        issue_in(1, 1)
        chunk(0, 0, False, True)
        chunk(1, 1, True, True)

        def loop_t(t, carry):
            k = 3 * t + 2
            chunk(2, k, True, True)
            chunk(0, k + 1, True, True)
            chunk(1, k + 2, True, True)
            return carry

        lax.fori_loop(0, 7, loop_t, 0)
        chunk(2, 23, True, True)
        chunk(0, 24, False, False)
        chunk(1, 25, False, False)
        wait_out(2)
        wait_out(0)
        wait_out(1)

        pltpu.sync_copy(big.at[pl.ds(0, 312)], y.at[pl.ds(r0, 312), col])

    plsc.subcore_barrier()

    qn = 496 + 8 * c
    q0 = 112 + 496 * c
    qa = q0 + jnp.minimum(32 * s, qn - 32)

    for h in range(2):
        col = pl.ds(h * H, H)
        pltpu.sync_copy(y.at[pl.ds(10 * qa, 328), col], big.at[pl.ds(0, 328)])
        pltpu.sync_copy(xt.at[pl.ds(qa, 32), col], obuf)

        def l3body(g, carry):
            _fold_group(rbuf, g, obuf, g, big, 1 + 10 * g)
            return carry
        lax.fori_loop(0, 32, l3body, 0)

        @pl.when(jnp.logical_and(c == 1, s == 0))
        def _special():
            pltpu.sync_copy(xt.at[pl.ds(6080, 48), col], buf_b.at[pl.ds(0, 48)])
            for t in range(3):
                a_t = (60808, 60904, 61008)[t]
                d_t = (60811 + 100 * t) - a_t
                sz = (104, 112, 104)[t]
                pltpu.sync_copy(xt.at[pl.ds(a_t, sz), col],
                                buf_a.at[pl.ds(0, sz)])
                for v in range(8):
                    sl = pl.ds(v * 16, 16)
                    rbuf[t, sl] = obuf[t, sl]

                def pbody(p, carry):
                    for v in range(8):
                        sl = pl.ds(v * 16, 16)
                        pv = buf_b[1 + 10 * t + p, sl]
                        for k in range(10):
                            pv = jnp.maximum(pv, buf_a[d_t + 10 * p + k, sl])
                        rbuf[t, sl] = jnp.maximum(rbuf[t, sl], pv)
                    return carry
                lax.fori_loop(0, 10, pbody, 0)

        pltpu.sync_copy(rbuf, y.at[pl.ds(qa, 32), col])


def _k2_body(xt, y, cbuf, xbuf, obuf, sbuf_s, sbuf_l, sbuf_t, fixb):
    c = lax.axis_index("c")
    s = lax.axis_index("s")
    w = c * 16 + s

    del w
    @pl.when(s == 0)
    def _w0():
        for h in range(1):
            col = pl.ds(pl.multiple_of(c * H, H), H)

            pltpu.sync_copy(xt.at[pl.ds(11096, 16), col], sbuf_s)
            pltpu.sync_copy(xt.at[pl.ds(110960, 151), col],
                            sbuf_l.at[pl.ds(0, 151)])

            def pt(p, carry):
                _fold_group(sbuf_s, p, sbuf_s, p, sbuf_l, 1 + 10 * p)
                return carry
            lax.fori_loop(0, 15, pt, 0)
            pltpu.sync_copy(sbuf_s, y.at[pl.ds(11096, 16), col])

            pltpu.sync_copy(xt.at[pl.ds(11104, 24), col], sbuf_t)
            pltpu.sync_copy(sbuf_t.at[pl.ds(8, 16)],
                            y.at[pl.ds(11112, 16), col])
            pltpu.sync_copy(sbuf_l.at[pl.ds(8, 136)],
                            y.at[pl.ds(110968, 136), col])
            pltpu.sync_copy(sbuf_l.at[pl.ds(144, 7)],
                            y.at[pl.ds(111104, 7), col])

            pltpu.sync_copy(y.at[pl.ds(11088, 8), col], fixb.at[pl.ds(8, 8)])
            pltpu.sync_copy(y.at[pl.ds(1104, 8), col], fixb.at[pl.ds(0, 8)])
            pltpu.sync_copy(xt.at[pl.ds(1104, 8), col], fixb.at[pl.ds(16, 8)])
            for v in range(8):
                sl = pl.ds(v * 16, 16)
                m = fixb[16 + 5, sl]
                for k in range(5):
                    m = jnp.maximum(m, fixb[8 + 3 + k, sl])
                for k in range(5):
                    m = jnp.maximum(m, sbuf_s[k, sl])
                fixb[5, sl] = m
                m = fixb[16 + 6, sl]
                for k in range(10):
                    m = jnp.maximum(m, sbuf_s[5 + k, sl])
                fixb[6, sl] = m
                m = jnp.maximum(fixb[16 + 7, sl], sbuf_s[15, sl])
                for k in range(9):
                    m = jnp.maximum(m, sbuf_t[8 + k, sl])
                fixb[7, sl] = m
            pltpu.sync_copy(fixb.at[pl.ds(0, 8)], y.at[pl.ds(1104, 8), col])

            pltpu.sync_copy(xt.at[pl.ds(0, 112), col], xbuf)

            pltpu.sync_copy(y.at[pl.ds(608, 520), col], cbuf)

            def l2b(g, carry):
                _fold_group(obuf, 61 + g, xbuf, 61 + g, cbuf, 3 + 10 * g)
                return carry
            lax.fori_loop(0, 50, l2b, 0)
            _fold_group(obuf, 111, xbuf, 111, cbuf, 503)

            pltpu.sync_copy(y.at[pl.ds(104, 512), col], cbuf.at[pl.ds(0, 512)])

            def l2a(g, carry):
                _fold_group(obuf, 11 + g, xbuf, 11 + g, cbuf, 7 + 10 * g)
                return carry
            lax.fori_loop(1, 50, l2a, 0)
            for v in range(8):
                sl = pl.ds(v * 16, 16)
                m = jnp.maximum(xbuf[11, sl], obuf[111, sl])
                for k in range(9):
                    m = jnp.maximum(m, cbuf[8 + k, sl])
                obuf[11, sl] = m

            def l1(r, carry):
                _fold_group(obuf, r, xbuf, r, obuf, 10 * r + 1)
                return carry
            lax.fori_loop(1, 11, l1, 0)

            _fold_group(obuf, 0, xbuf, 0, obuf, 1)

            pltpu.sync_copy(obuf, y.at[pl.ds(0, 112), col])


def kernel(x, level_parents, level_children):
    del level_parents, level_children
    xt = x.T

    k1 = pl.kernel(
        _k1_body, out_type=(),
        mesh=plsc.VectorSubcoreMesh(**_MESH),
        scratch_types=[
            pltpu.VMEM((128, H), jnp.float32),
            pltpu.VMEM((128, H), jnp.float32),
            pltpu.VMEM((128, H), jnp.float32),
            pltpu.VMEM((328, H), jnp.float32),
            pltpu.VMEM((32, H), jnp.float32),
            pltpu.VMEM((32, H), jnp.float32),
            pltpu.SemaphoreType.DMA,
            pltpu.SemaphoreType.DMA,
            pltpu.SemaphoreType.DMA,
            pltpu.SemaphoreType.DMA,
            pltpu.SemaphoreType.DMA,
            pltpu.SemaphoreType.DMA,
        ],
        compiler_params=_CP,
    )
    k2 = pl.kernel(
        _k2_body, out_type=(),
        mesh=plsc.VectorSubcoreMesh(**_MESH),
        scratch_types=[
            pltpu.VMEM((520, H), jnp.float32),
            pltpu.VMEM((112, H), jnp.float32),
            pltpu.VMEM((112, H), jnp.float32),
            pltpu.VMEM((16, H), jnp.float32),
            pltpu.VMEM((152, H), jnp.float32),
            pltpu.VMEM((24, H), jnp.float32),
            pltpu.VMEM((24, H), jnp.float32),
        ],
        compiler_params=_CP,
    )

    y_ref = jax.new_ref(lax.empty((N, B), jnp.float32))
    k1(xt, y_ref)
    k2(xt, y_ref)
    return y_ref[...].T
